# Initial kernel scaffold; baseline (speedup 1.0000x reference)
import jax
import jax.numpy as jnp
from jax.experimental import pallas as pl


def kernel(inputs, temporal_embedding, spectoral_embedding, ln_scale, ln_bias):
    raise NotImplementedError("write your pallas kernel here")


# speedup vs baseline: 2.5412x; 2.5412x over previous
import jax
import jax.numpy as jnp
from jax.experimental import pallas as pl

T, S = 256, 16
HIDDEN = 1024
BATCH = 4
TT = 32


def _body(t_ref, s_ref, g_ref, b_ref, x_ref, o_ref):
    pos = t_ref[...][:, None, :] + s_ref[...][None, :, :]
    mean = jnp.mean(pos, axis=-1, keepdims=True)
    cen = pos - mean
    var = jnp.mean(cen * cen, axis=-1, keepdims=True)
    pos = cen * jax.lax.rsqrt(var + 1e-6) * g_ref[0] + b_ref[0]
    o_ref[...] = x_ref[...] + pos[None]


def kernel(inputs, temporal_embedding, spectoral_embedding, ln_scale, ln_bias):
    x = inputs.reshape(BATCH, T, S, HIDDEN)
    out = pl.pallas_call(
        _body,
        grid=(T // TT,),
        in_specs=[
            pl.BlockSpec((TT, HIDDEN), lambda i: (i, 0)),
            pl.BlockSpec((S, HIDDEN), lambda i: (0, 0)),
            pl.BlockSpec((1, HIDDEN), lambda i: (0, 0)),
            pl.BlockSpec((1, HIDDEN), lambda i: (0, 0)),
            pl.BlockSpec((BATCH, TT, S, HIDDEN), lambda i: (0, i, 0, 0)),
        ],
        out_specs=pl.BlockSpec((BATCH, TT, S, HIDDEN), lambda i: (0, i, 0, 0)),
        out_shape=jax.ShapeDtypeStruct((BATCH, T, S, HIDDEN), jnp.float32),
    )(
        temporal_embedding,
        spectoral_embedding,
        ln_scale.reshape(1, HIDDEN),
        ln_bias.reshape(1, HIDDEN),
        x,
    )
    return out.reshape(BATCH, 1, T * S, HIDDEN)
